# baseline (speedup 1.0000x reference)
import functools

import jax
import jax.numpy as jnp
from jax.experimental import pallas as pl
from jax.experimental.pallas import tpu as pltpu
from jax.experimental.pallas import tpu_sc as plsc

NUM_EXPERTS = 8
TOP_K = 2
HIDDEN = 768
INTER = 512
TOKENS = 2048
PAIRS = TOKENS * TOP_K
BM = 256
NUM_BLOCKS = PAIRS // BM + NUM_EXPERTS - 1
PAD_ROWS = NUM_BLOCKS * BM
NW = 32
TPW = TOKENS // NW

def _sc_mesh():
    return plsc.VectorSubcoreMesh(core_axis_name="c", subcore_axis_name="s")


def _routing_body(lt_ref, dest_ref, w_ref, bmap_ref):
    lt = lt_ref[...]
    iota_e = jax.lax.broadcasted_iota(jnp.int32, (NUM_EXPERTS, TOKENS), 0)
    m0 = jnp.max(lt, axis=0, keepdims=True)
    e0 = jnp.min(jnp.where(lt >= m0, iota_e, NUM_EXPERTS), axis=0, keepdims=True)
    sel0 = iota_e == e0
    lt2 = jnp.where(sel0, -jnp.inf, lt)
    m1 = jnp.max(lt2, axis=0, keepdims=True)
    e1 = jnp.min(jnp.where(lt2 >= m1, iota_e, NUM_EXPERTS), axis=0, keepdims=True)
    sel1 = iota_e == e1

    t = jnp.exp(m1 - m0)
    w0 = 1.0 / (1.0 + t)
    w1 = 1.0 - w0

    mi = sel0.astype(jnp.int32) + sel1.astype(jnp.int32)
    c = mi
    sh = 1
    while sh < TOKENS:
        z = jnp.zeros((NUM_EXPERTS, sh), jnp.int32)
        c = c + jnp.concatenate([z, c[:, : TOKENS - sh]], axis=1)
        sh *= 2
    counts = c[:, TOKENS - 1 : TOKENS]
    cex = c - mi
    pc = ((counts + (BM - 1)) // BM) * BM
    seg = pc
    sh = 1
    while sh < NUM_EXPERTS:
        z = jnp.zeros((sh, 1), jnp.int32)
        seg = seg + jnp.concatenate([z, seg[: NUM_EXPERTS - sh]], axis=0)
        sh *= 2
    seg = seg - pc
    slot = jnp.broadcast_to(seg, (NUM_EXPERTS, TOKENS)) + cex
    dest0 = jnp.sum(jnp.where(sel0, slot, 0), axis=0, keepdims=True)
    dest1 = jnp.sum(jnp.where(sel1, slot, 0), axis=0, keepdims=True)

    endblk = (seg + pc) // BM
    biota = jax.lax.broadcasted_iota(jnp.int32, (1, 128), 1)
    be = jnp.zeros((1, 128), jnp.int32)
    for e in range(NUM_EXPERTS):
        be = be + (biota >= endblk[e : e + 1, :]).astype(jnp.int32)
    dest_ref[...] = jnp.concatenate([dest0, dest1], axis=0)
    w_ref[...] = jnp.concatenate([w0, w1], axis=0)
    bmap_ref[...] = jnp.minimum(be, NUM_EXPERTS - 1)


def _routing(router_logits):
    return pl.pallas_call(
        _routing_body,
        out_shape=(
            jax.ShapeDtypeStruct((TOP_K, TOKENS), jnp.int32),
            jax.ShapeDtypeStruct((TOP_K, TOKENS), jnp.float32),
            jax.ShapeDtypeStruct((1, 128), jnp.int32),
        ),
    )(router_logits.T)


def _sc_scatter_rows(hid, d0, d1):
    @functools.partial(
        pl.kernel,
        out_type=jax.ShapeDtypeStruct((PAD_ROWS, HIDDEN), jnp.float32),
        mesh=_sc_mesh(),
        scratch_types=[
            pltpu.VMEM((1, TPW), jnp.int32),
            pltpu.VMEM((1, TPW), jnp.int32),
            pltpu.VMEM((TPW, HIDDEN), jnp.float32),
            pltpu.SemaphoreType.DMA,
            pltpu.SemaphoreType.DMA,
            pltpu.SemaphoreType.DMA,
        ],
    )
    def body(hid_hbm, d0_hbm, d1_hbm, xpad_hbm, i0_v, i1_v, rows_v, s0, s1, s2):
        wid = jax.lax.axis_index("s") * 2 + jax.lax.axis_index("c")
        base = wid * TPW
        c0 = pltpu.async_copy(d0_hbm.at[pl.ds(base, TPW)], i0_v.at[0], s0)
        c1 = pltpu.async_copy(d1_hbm.at[pl.ds(base, TPW)], i1_v.at[0], s1)
        c2 = pltpu.async_copy(hid_hbm.at[pl.ds(base, TPW)], rows_v, s2)
        c0.wait()
        c1.wait()
        c2.wait()
        c3 = pltpu.async_copy(rows_v, xpad_hbm.at[i0_v.at[0]], s0)
        c4 = pltpu.async_copy(rows_v, xpad_hbm.at[i1_v.at[0]], s1)
        c3.wait()
        c4.wait()

    return body(hid, d0, d1)


def _gmm_body(bmap_ref, x_ref, gw_ref, uw_ref, dw_ref, o_ref):
    x = x_ref[...].astype(jnp.bfloat16)
    g = jnp.dot(x, gw_ref[0], preferred_element_type=jnp.float32)
    u = jnp.dot(x, uw_ref[0], preferred_element_type=jnp.float32)
    a = ((g / (1.0 + jnp.exp(-g))) * u).astype(jnp.bfloat16)
    o_ref[...] = jnp.dot(a, dw_ref[0], preferred_element_type=jnp.float32)


def _grouped_ffn(bmap, x_pad, gate_w, up_w, down_w):
    grid_spec = pltpu.PrefetchScalarGridSpec(
        num_scalar_prefetch=1,
        grid=(NUM_BLOCKS,),
        in_specs=[
            pl.BlockSpec((BM, HIDDEN), lambda i, m: (i, 0)),
            pl.BlockSpec((1, HIDDEN, INTER), lambda i, m: (m[i], 0, 0)),
            pl.BlockSpec((1, HIDDEN, INTER), lambda i, m: (m[i], 0, 0)),
            pl.BlockSpec((1, INTER, HIDDEN), lambda i, m: (m[i], 0, 0)),
        ],
        out_specs=pl.BlockSpec((BM, HIDDEN), lambda i, m: (i, 0)),
    )
    return pl.pallas_call(
        _gmm_body,
        grid_spec=grid_spec,
        out_shape=jax.ShapeDtypeStruct((PAD_ROWS, HIDDEN), jnp.float32),
    )(bmap, x_pad, gate_w.astype(jnp.bfloat16), up_w.astype(jnp.bfloat16),
      down_w.astype(jnp.bfloat16))


def _sc_gather_combine(ypad, d0, d1, w0r, w1r):
    @functools.partial(
        pl.kernel,
        out_type=jax.ShapeDtypeStruct((TOKENS, HIDDEN), jnp.float32),
        mesh=_sc_mesh(),
        scratch_types=[
            pltpu.VMEM((1, TPW), jnp.int32),
            pltpu.VMEM((1, TPW), jnp.int32),
            pltpu.VMEM((TPW, 16), jnp.float32),
            pltpu.VMEM((TPW, 16), jnp.float32),
            pltpu.VMEM((TPW, HIDDEN), jnp.float32),
            pltpu.VMEM((TPW, HIDDEN), jnp.float32),
            pltpu.SemaphoreType.DMA,
            pltpu.SemaphoreType.DMA,
            pltpu.SemaphoreType.DMA,
            pltpu.SemaphoreType.DMA,
        ],
    )
    def body(ypad_hbm, d0_hbm, d1_hbm, w0_hbm, w1_hbm, out_hbm,
             i0_v, i1_v, w0_v, w1_v, r0_v, r1_v, s0, s1, s2, s3):
        wid = jax.lax.axis_index("s") * 2 + jax.lax.axis_index("c")
        base = wid * TPW
        c0 = pltpu.async_copy(d0_hbm.at[pl.ds(base, TPW)], i0_v.at[0], s0)
        c1 = pltpu.async_copy(d1_hbm.at[pl.ds(base, TPW)], i1_v.at[0], s1)
        c2 = pltpu.async_copy(w0_hbm.at[pl.ds(base, TPW)], w0_v, s2)
        c3 = pltpu.async_copy(w1_hbm.at[pl.ds(base, TPW)], w1_v, s3)
        c0.wait()
        c1.wait()
        g0 = pltpu.async_copy(ypad_hbm.at[i0_v.at[0]], r0_v, s0)
        g1 = pltpu.async_copy(ypad_hbm.at[i1_v.at[0]], r1_v, s1)
        c2.wait()
        c3.wait()
        g0.wait()
        g1.wait()

        @pl.loop(0, TPW)
        def _row(j):
            wv0 = w0_v[j]
            wv1 = w1_v[j]
            for c in range(0, HIDDEN, 16):
                s = (j, pl.ds(c, 16))
                r0_v[s] = r0_v[s] * wv0 + r1_v[s] * wv1

        pltpu.sync_copy(r0_v, out_hbm.at[pl.ds(base, TPW)])

    return body(ypad, d0, d1, w0r, w1r)


def kernel(hidden_states, router_logits, gate_w, up_w, down_w):
    dest01, w01, bmap = _routing(router_logits)
    x_pad = _sc_scatter_rows(hidden_states, dest01[0], dest01[1])
    y_pad = _grouped_ffn(bmap[0], x_pad, gate_w, up_w, down_w)
    w0r = jnp.broadcast_to(w01[0][:, None], (TOKENS, 16))
    w1r = jnp.broadcast_to(w01[1][:, None], (TOKENS, 16))
    return _sc_gather_combine(y_pad, dest01[0], dest01[1], w0r, w1r)


# speedup vs baseline: 1.1627x; 1.1627x over previous
import functools

import jax
import jax.numpy as jnp
from jax.experimental import pallas as pl
from jax.experimental.pallas import tpu as pltpu
from jax.experimental.pallas import tpu_sc as plsc

NUM_EXPERTS = 8
TOP_K = 2
HIDDEN = 768
INTER = 512
TOKENS = 2048
PAIRS = TOKENS * TOP_K
BM = 256
NUM_BLOCKS = PAIRS // BM + NUM_EXPERTS - 1
PAD_ROWS = NUM_BLOCKS * BM
NW = 32
TPW = TOKENS // NW

def _sc_mesh():
    return plsc.VectorSubcoreMesh(core_axis_name="c", subcore_axis_name="s")


def _routing_body(lt_ref, dest_ref, w_ref, bmap_ref):
    lt = lt_ref[...]
    iota_e = jax.lax.broadcasted_iota(jnp.int32, (NUM_EXPERTS, TOKENS), 0)
    m0 = jnp.max(lt, axis=0, keepdims=True)
    e0 = jnp.min(jnp.where(lt >= m0, iota_e, NUM_EXPERTS), axis=0, keepdims=True)
    sel0 = iota_e == e0
    lt2 = jnp.where(sel0, -jnp.inf, lt)
    m1 = jnp.max(lt2, axis=0, keepdims=True)
    e1 = jnp.min(jnp.where(lt2 >= m1, iota_e, NUM_EXPERTS), axis=0, keepdims=True)
    sel1 = iota_e == e1

    t = jnp.exp(m1 - m0)
    w0 = 1.0 / (1.0 + t)
    w1 = 1.0 - w0

    mi = sel0.astype(jnp.int32) + sel1.astype(jnp.int32)
    c = mi
    sh = 1
    while sh < TOKENS:
        z = jnp.zeros((NUM_EXPERTS, sh), jnp.int32)
        c = c + jnp.concatenate([z, c[:, : TOKENS - sh]], axis=1)
        sh *= 2
    counts = c[:, TOKENS - 1 : TOKENS]
    cex = c - mi
    pc = ((counts + (BM - 1)) // BM) * BM
    seg = pc
    sh = 1
    while sh < NUM_EXPERTS:
        z = jnp.zeros((sh, 1), jnp.int32)
        seg = seg + jnp.concatenate([z, seg[: NUM_EXPERTS - sh]], axis=0)
        sh *= 2
    seg = seg - pc
    slot = jnp.broadcast_to(seg, (NUM_EXPERTS, TOKENS)) + cex
    dest0 = jnp.sum(jnp.where(sel0, slot, 0), axis=0, keepdims=True)
    dest1 = jnp.sum(jnp.where(sel1, slot, 0), axis=0, keepdims=True)

    endblk = (seg + pc) // BM
    biota = jax.lax.broadcasted_iota(jnp.int32, (1, 128), 1)
    be = jnp.zeros((1, 128), jnp.int32)
    for e in range(NUM_EXPERTS):
        be = be + (biota >= endblk[e : e + 1, :]).astype(jnp.int32)
    dest_ref[...] = jnp.concatenate([dest0, dest1], axis=0)
    w_ref[...] = jnp.concatenate([w0, w1], axis=0)
    bmap_ref[...] = jnp.minimum(be, NUM_EXPERTS - 1)


def _routing(router_logits):
    return pl.pallas_call(
        _routing_body,
        out_shape=(
            jax.ShapeDtypeStruct((TOP_K, TOKENS), jnp.int32),
            jax.ShapeDtypeStruct((TOP_K, TOKENS), jnp.float32),
            jax.ShapeDtypeStruct((1, 128), jnp.int32),
        ),
    )(router_logits.T)


def _sc_scatter_rows(hid, d0, d1):
    @functools.partial(
        pl.kernel,
        out_type=jax.ShapeDtypeStruct((PAD_ROWS, HIDDEN), jnp.float32),
        mesh=_sc_mesh(),
        scratch_types=[
            pltpu.VMEM((1, TPW), jnp.int32),
            pltpu.VMEM((1, TPW), jnp.int32),
            pltpu.VMEM((TPW, HIDDEN), jnp.float32),
            pltpu.SemaphoreType.DMA,
            pltpu.SemaphoreType.DMA,
            pltpu.SemaphoreType.DMA,
        ],
    )
    def body(hid_hbm, d0_hbm, d1_hbm, xpad_hbm, i0_v, i1_v, rows_v, s0, s1, s2):
        wid = jax.lax.axis_index("s") * 2 + jax.lax.axis_index("c")
        base = wid * TPW
        c0 = pltpu.async_copy(d0_hbm.at[pl.ds(base, TPW)], i0_v.at[0], s0)
        c1 = pltpu.async_copy(d1_hbm.at[pl.ds(base, TPW)], i1_v.at[0], s1)
        c2 = pltpu.async_copy(hid_hbm.at[pl.ds(base, TPW)], rows_v, s2)
        c0.wait()
        c1.wait()
        c2.wait()
        c3 = pltpu.async_copy(rows_v, xpad_hbm.at[i0_v.at[0]], s0)
        c4 = pltpu.async_copy(rows_v, xpad_hbm.at[i1_v.at[0]], s1)
        c3.wait()
        c4.wait()

    return body(hid, d0, d1)


def _gmm_body(bmap_ref, x_ref, gw_ref, uw_ref, dw_ref, o_ref):
    x = x_ref[...].astype(jnp.bfloat16)
    g = jnp.dot(x, gw_ref[0].astype(jnp.bfloat16), preferred_element_type=jnp.float32)
    u = jnp.dot(x, uw_ref[0].astype(jnp.bfloat16), preferred_element_type=jnp.float32)
    a = ((g / (1.0 + jnp.exp(-g))) * u).astype(jnp.bfloat16)
    o_ref[...] = jnp.dot(a, dw_ref[0].astype(jnp.bfloat16), preferred_element_type=jnp.float32)


def _grouped_ffn(bmap, x_pad, gate_w, up_w, down_w):
    grid_spec = pltpu.PrefetchScalarGridSpec(
        num_scalar_prefetch=1,
        grid=(NUM_BLOCKS,),
        in_specs=[
            pl.BlockSpec((BM, HIDDEN), lambda i, m: (i, 0)),
            pl.BlockSpec((1, HIDDEN, INTER), lambda i, m: (m[i], 0, 0)),
            pl.BlockSpec((1, HIDDEN, INTER), lambda i, m: (m[i], 0, 0)),
            pl.BlockSpec((1, INTER, HIDDEN), lambda i, m: (m[i], 0, 0)),
        ],
        out_specs=pl.BlockSpec((BM, HIDDEN), lambda i, m: (i, 0)),
    )
    return pl.pallas_call(
        _gmm_body,
        grid_spec=grid_spec,
        out_shape=jax.ShapeDtypeStruct((PAD_ROWS, HIDDEN), jnp.float32),
    )(bmap, x_pad, gate_w, up_w, down_w)


def _sc_gather_combine(ypad, d0, d1, w0r, w1r):
    @functools.partial(
        pl.kernel,
        out_type=jax.ShapeDtypeStruct((TOKENS, HIDDEN), jnp.float32),
        mesh=_sc_mesh(),
        scratch_types=[
            pltpu.VMEM((1, TPW), jnp.int32),
            pltpu.VMEM((1, TPW), jnp.int32),
            pltpu.VMEM((TPW, 16), jnp.float32),
            pltpu.VMEM((TPW, 16), jnp.float32),
            pltpu.VMEM((TPW, HIDDEN), jnp.float32),
            pltpu.VMEM((TPW, HIDDEN), jnp.float32),
            pltpu.SemaphoreType.DMA,
            pltpu.SemaphoreType.DMA,
            pltpu.SemaphoreType.DMA,
            pltpu.SemaphoreType.DMA,
        ],
    )
    def body(ypad_hbm, d0_hbm, d1_hbm, w0_hbm, w1_hbm, out_hbm,
             i0_v, i1_v, w0_v, w1_v, r0_v, r1_v, s0, s1, s2, s3):
        wid = jax.lax.axis_index("s") * 2 + jax.lax.axis_index("c")
        base = wid * TPW
        c0 = pltpu.async_copy(d0_hbm.at[pl.ds(base, TPW)], i0_v.at[0], s0)
        c1 = pltpu.async_copy(d1_hbm.at[pl.ds(base, TPW)], i1_v.at[0], s1)
        c2 = pltpu.async_copy(w0_hbm.at[pl.ds(base, TPW)], w0_v, s2)
        c3 = pltpu.async_copy(w1_hbm.at[pl.ds(base, TPW)], w1_v, s3)
        c0.wait()
        c1.wait()
        g0 = pltpu.async_copy(ypad_hbm.at[i0_v.at[0]], r0_v, s0)
        g1 = pltpu.async_copy(ypad_hbm.at[i1_v.at[0]], r1_v, s1)
        c2.wait()
        c3.wait()
        g0.wait()
        g1.wait()

        @pl.loop(0, TPW)
        def _row(j):
            wv0 = w0_v[j]
            wv1 = w1_v[j]
            for c in range(0, HIDDEN, 16):
                s = (j, pl.ds(c, 16))
                r0_v[s] = r0_v[s] * wv0 + r1_v[s] * wv1

        pltpu.sync_copy(r0_v, out_hbm.at[pl.ds(base, TPW)])

    return body(ypad, d0, d1, w0r, w1r)


def kernel(hidden_states, router_logits, gate_w, up_w, down_w):
    dest01, w01, bmap = _routing(router_logits)
    x_pad = _sc_scatter_rows(hidden_states, dest01[0], dest01[1])
    y_pad = _grouped_ffn(bmap[0], x_pad, gate_w, up_w, down_w)
    w0r = jnp.broadcast_to(w01[0][:, None], (TOKENS, 16))
    w1r = jnp.broadcast_to(w01[1][:, None], (TOKENS, 16))
    return _sc_gather_combine(y_pad, dest01[0], dest01[1], w0r, w1r)


# speedup vs baseline: 1.1924x; 1.0256x over previous
import functools

import jax
import jax.numpy as jnp
from jax.experimental import pallas as pl
from jax.experimental.pallas import tpu as pltpu
from jax.experimental.pallas import tpu_sc as plsc

NUM_EXPERTS = 8
TOP_K = 2
HIDDEN = 768
INTER = 512
TOKENS = 2048
PAIRS = TOKENS * TOP_K
BM = 256
NUM_BLOCKS = PAIRS // BM + NUM_EXPERTS - 1
PAD_ROWS = NUM_BLOCKS * BM
NW = 32
TPW = TOKENS // NW

def _sc_mesh():
    return plsc.VectorSubcoreMesh(core_axis_name="c", subcore_axis_name="s")


def _routing_body(lt_ref, d0_ref, d1_ref, w0_ref, w1_ref, bmap_ref):
    lt = lt_ref[...]
    iota_e = jax.lax.broadcasted_iota(jnp.int32, (NUM_EXPERTS, TOKENS), 0)
    m0 = jnp.max(lt, axis=0, keepdims=True)
    e0 = jnp.min(jnp.where(lt >= m0, iota_e, NUM_EXPERTS), axis=0, keepdims=True)
    sel0 = iota_e == e0
    lt2 = jnp.where(sel0, -jnp.inf, lt)
    m1 = jnp.max(lt2, axis=0, keepdims=True)
    e1 = jnp.min(jnp.where(lt2 >= m1, iota_e, NUM_EXPERTS), axis=0, keepdims=True)
    sel1 = iota_e == e1

    t = jnp.exp(m1 - m0)
    w0 = 1.0 / (1.0 + t)
    w1 = 1.0 - w0

    mi = sel0.astype(jnp.int32) + sel1.astype(jnp.int32)
    c = mi
    sh = 1
    while sh < TOKENS:
        z = jnp.zeros((NUM_EXPERTS, sh), jnp.int32)
        c = c + jnp.concatenate([z, c[:, : TOKENS - sh]], axis=1)
        sh *= 2
    counts = c[:, TOKENS - 1 : TOKENS]
    cex = c - mi
    pc = ((counts + (BM - 1)) // BM) * BM
    seg = pc
    sh = 1
    while sh < NUM_EXPERTS:
        z = jnp.zeros((sh, 1), jnp.int32)
        seg = seg + jnp.concatenate([z, seg[: NUM_EXPERTS - sh]], axis=0)
        sh *= 2
    seg = seg - pc
    slot = jnp.broadcast_to(seg, (NUM_EXPERTS, TOKENS)) + cex
    dest0 = jnp.sum(jnp.where(sel0, slot, 0), axis=0, keepdims=True)
    dest1 = jnp.sum(jnp.where(sel1, slot, 0), axis=0, keepdims=True)

    endblk = (seg + pc) // BM
    biota = jax.lax.broadcasted_iota(jnp.int32, (1, 128), 1)
    be = jnp.zeros((1, 128), jnp.int32)
    for e in range(NUM_EXPERTS):
        be = be + (biota >= endblk[e : e + 1, :]).astype(jnp.int32)
    d0_ref[...] = dest0
    d1_ref[...] = dest1
    w0_ref[...] = jnp.broadcast_to(w0.T, (TOKENS, 16))
    w1_ref[...] = jnp.broadcast_to(w1.T, (TOKENS, 16))
    bmap_ref[...] = jnp.minimum(be, NUM_EXPERTS - 1)


def _routing(router_logits):
    return pl.pallas_call(
        _routing_body,
        out_shape=(
            jax.ShapeDtypeStruct((1, TOKENS), jnp.int32),
            jax.ShapeDtypeStruct((1, TOKENS), jnp.int32),
            jax.ShapeDtypeStruct((TOKENS, 16), jnp.float32),
            jax.ShapeDtypeStruct((TOKENS, 16), jnp.float32),
            jax.ShapeDtypeStruct((1, 128), jnp.int32),
        ),
    )(router_logits.T)


def _sc_scatter_rows(hid, d0, d1):
    @functools.partial(
        pl.kernel,
        out_type=jax.ShapeDtypeStruct((PAD_ROWS, HIDDEN), jnp.float32),
        mesh=_sc_mesh(),
        scratch_types=[
            pltpu.VMEM((1, TPW), jnp.int32),
            pltpu.VMEM((1, TPW), jnp.int32),
            pltpu.VMEM((TPW, HIDDEN), jnp.float32),
            pltpu.SemaphoreType.DMA,
            pltpu.SemaphoreType.DMA,
            pltpu.SemaphoreType.DMA,
        ],
    )
    def body(hid_hbm, d0_hbm, d1_hbm, xpad_hbm, i0_v, i1_v, rows_v, s0, s1, s2):
        wid = jax.lax.axis_index("s") * 2 + jax.lax.axis_index("c")
        base = wid * TPW
        c0 = pltpu.async_copy(d0_hbm.at[pl.ds(base, TPW)], i0_v.at[0], s0)
        c1 = pltpu.async_copy(d1_hbm.at[pl.ds(base, TPW)], i1_v.at[0], s1)
        c2 = pltpu.async_copy(hid_hbm.at[pl.ds(base, TPW)], rows_v, s2)
        c0.wait()
        c1.wait()
        c2.wait()
        c3 = pltpu.async_copy(rows_v, xpad_hbm.at[i0_v.at[0]], s0)
        c4 = pltpu.async_copy(rows_v, xpad_hbm.at[i1_v.at[0]], s1)
        c3.wait()
        c4.wait()

    return body(hid, d0, d1)


def _gmm_body(bmap_ref, x_ref, gw_ref, uw_ref, dw_ref, o_ref):
    x = x_ref[...]
    g = jnp.dot(x, gw_ref[0], preferred_element_type=jnp.float32)
    u = jnp.dot(x, uw_ref[0], preferred_element_type=jnp.float32)
    a = (g / (1.0 + jnp.exp(-g))) * u
    o_ref[...] = jnp.dot(a, dw_ref[0], preferred_element_type=jnp.float32)


def _grouped_ffn(bmap, x_pad, gate_w, up_w, down_w):
    grid_spec = pltpu.PrefetchScalarGridSpec(
        num_scalar_prefetch=1,
        grid=(NUM_BLOCKS,),
        in_specs=[
            pl.BlockSpec((BM, HIDDEN), lambda i, m: (i, 0)),
            pl.BlockSpec((1, HIDDEN, INTER), lambda i, m: (m[i], 0, 0)),
            pl.BlockSpec((1, HIDDEN, INTER), lambda i, m: (m[i], 0, 0)),
            pl.BlockSpec((1, INTER, HIDDEN), lambda i, m: (m[i], 0, 0)),
        ],
        out_specs=pl.BlockSpec((BM, HIDDEN), lambda i, m: (i, 0)),
    )
    return pl.pallas_call(
        _gmm_body,
        grid_spec=grid_spec,
        out_shape=jax.ShapeDtypeStruct((PAD_ROWS, HIDDEN), jnp.float32),
    )(bmap, x_pad, gate_w, up_w, down_w)


def _sc_gather_combine(ypad, d0, d1, w0r, w1r):
    @functools.partial(
        pl.kernel,
        out_type=jax.ShapeDtypeStruct((TOKENS, HIDDEN), jnp.float32),
        mesh=_sc_mesh(),
        scratch_types=[
            pltpu.VMEM((1, TPW), jnp.int32),
            pltpu.VMEM((1, TPW), jnp.int32),
            pltpu.VMEM((TPW, 16), jnp.float32),
            pltpu.VMEM((TPW, 16), jnp.float32),
            pltpu.VMEM((TPW, HIDDEN), jnp.float32),
            pltpu.VMEM((TPW, HIDDEN), jnp.float32),
            pltpu.SemaphoreType.DMA,
            pltpu.SemaphoreType.DMA,
            pltpu.SemaphoreType.DMA,
            pltpu.SemaphoreType.DMA,
        ],
    )
    def body(ypad_hbm, d0_hbm, d1_hbm, w0_hbm, w1_hbm, out_hbm,
             i0_v, i1_v, w0_v, w1_v, r0_v, r1_v, s0, s1, s2, s3):
        wid = jax.lax.axis_index("s") * 2 + jax.lax.axis_index("c")
        base = wid * TPW
        c0 = pltpu.async_copy(d0_hbm.at[pl.ds(base, TPW)], i0_v.at[0], s0)
        c1 = pltpu.async_copy(d1_hbm.at[pl.ds(base, TPW)], i1_v.at[0], s1)
        c2 = pltpu.async_copy(w0_hbm.at[pl.ds(base, TPW)], w0_v, s2)
        c3 = pltpu.async_copy(w1_hbm.at[pl.ds(base, TPW)], w1_v, s3)
        c0.wait()
        c1.wait()
        g0 = pltpu.async_copy(ypad_hbm.at[i0_v.at[0]], r0_v, s0)
        g1 = pltpu.async_copy(ypad_hbm.at[i1_v.at[0]], r1_v, s1)
        c2.wait()
        c3.wait()
        g0.wait()
        g1.wait()

        @pl.loop(0, TPW)
        def _row(j):
            wv0 = w0_v[j]
            wv1 = w1_v[j]
            for c in range(0, HIDDEN, 16):
                s = (j, pl.ds(c, 16))
                r0_v[s] = r0_v[s] * wv0 + r1_v[s] * wv1

        pltpu.sync_copy(r0_v, out_hbm.at[pl.ds(base, TPW)])

    return body(ypad, d0, d1, w0r, w1r)


def kernel(hidden_states, router_logits, gate_w, up_w, down_w):
    d0, d1, w0r, w1r, bmap = _routing(router_logits)
    x_pad = _sc_scatter_rows(hidden_states, d0[0], d1[0])
    y_pad = _grouped_ffn(bmap[0], x_pad, gate_w, up_w, down_w)
    return _sc_gather_combine(y_pad, d0[0], d1[0], w0r, w1r)
